# baseline (speedup 1.0000x reference)
import jax
import jax.numpy as jnp
from jax import lax
from jax.experimental import pallas as pl
from jax.experimental.pallas import tpu as pltpu
from jax.experimental.pallas import tpu_sc as plsc

NUM_MASHUP = 100000
NUM_API = 1000
BATCH = 4096

_info = plsc.get_sparse_core_info()
_NC, _NS, _L = _info.num_cores, _info.num_subcores, _info.num_lanes
_NW = _NC * _NS
_BPW = BATCH // _NW
_CHUNK = 32
_NCHUNK = _BPW // _CHUNK
_NSLICE = -(-NUM_API // _L)
_TAIL = NUM_API - _L


def _sc_body(table_hbm, idx_hbm, out_hbm, idx_v, in_v, out_v, sem):
    wid = lax.axis_index("s") * _NC + lax.axis_index("c")
    base = wid * _BPW
    for c in range(_NCHUNK):
        pltpu.sync_copy(idx_hbm.at[pl.ds(base + c * _CHUNK, _CHUNK)], idx_v.at[c])
        pltpu.async_copy(table_hbm.at[idx_v.at[c]], in_v, sem).wait()

        def row_body(r, carry):
            offs = [j * _L for j in range(NUM_API // _L)] + [_TAIL]
            for o in offs:
                x = in_v[r, pl.ds(o, _L)]
                out_v[r, pl.ds(o, _L)] = 1.0 / (1.0 + jnp.exp(-x))
            return carry

        lax.fori_loop(0, _CHUNK, row_body, 0)
        pltpu.sync_copy(out_v, out_hbm.at[pl.ds(base + c * _CHUNK, _CHUNK)])


def kernel(m2a_mat, user_indices):
    f = pl.kernel(
        _sc_body,
        mesh=plsc.VectorSubcoreMesh(core_axis_name="c", subcore_axis_name="s"),
        out_type=jax.ShapeDtypeStruct((BATCH, NUM_API), jnp.float32),
        scratch_types=[
            pltpu.VMEM((_NCHUNK, _CHUNK), jnp.int32),
            pltpu.VMEM((_CHUNK, NUM_API), jnp.float32),
            pltpu.VMEM((_CHUNK, NUM_API), jnp.float32),
            pltpu.SemaphoreType.DMA,
        ],
        compiler_params=pltpu.CompilerParams(use_tc_tiling_on_sc=False),
    )
    return f(m2a_mat, user_indices.astype(jnp.int32))


# speedup vs baseline: 3.9537x; 3.9537x over previous
import jax
import jax.numpy as jnp
from jax import lax
from jax.experimental import pallas as pl
from jax.experimental.pallas import tpu as pltpu
from jax.experimental.pallas import tpu_sc as plsc

NUM_MASHUP = 100000
NUM_API = 1000
BATCH = 4096

_info = plsc.get_sparse_core_info()
_NC, _NS, _L = _info.num_cores, _info.num_subcores, _info.num_lanes
_NW = _NC * _NS
_BPW = BATCH // _NW
_CHUNK = 32
_NCHUNK = _BPW // _CHUNK
_COL7 = 896
_TAILW = NUM_API - _COL7


def _sigmoid(x):
    return 1.0 / (1.0 + jnp.exp(-x))


def _sc_body(table_hbm, idx_hbm, out_hbm, idx_v, in_v, tails, out_v, sem, sem2):
    wid = lax.axis_index("s") * _NC + lax.axis_index("c")
    base = wid * _BPW
    lanes = lax.iota(jnp.int32, _L)
    for c in range(_NCHUNK):
        pltpu.sync_copy(idx_hbm.at[pl.ds(base + c * _CHUNK, _CHUNK)], idx_v.at[c])
        main_copies = [
            pltpu.async_copy(
                table_hbm.at[idx_v.at[c], pl.ds(k * 128, 128)],
                in_v.at[:, pl.ds(k * 128, 128)],
                sem,
            )
            for k in range(_COL7 // 128)
        ]

        def tail_issue(r, carry):
            voff = pl.multiple_of((r // _L) * _L, _L)
            v = idx_v[c, pl.ds(voff, _L)]
            i = jnp.max(jnp.where(lanes == r % _L, v, 0))
            pltpu.async_copy(
                table_hbm.at[pl.ds(i, 1), pl.ds(_COL7, _TAILW)],
                tails.at[pl.ds(r, 1)],
                sem2,
            )
            return carry

        lax.fori_loop(0, _CHUNK, tail_issue, 0)

        for cp in main_copies:
            cp.wait()

        def main_compute(r, carry):
            for j in range(_COL7 // _L):
                x = in_v[r, pl.ds(j * _L, _L)]
                out_v[r, pl.ds(j * _L, _L)] = _sigmoid(x)
            return carry

        lax.fori_loop(0, _CHUNK, main_compute, 0)

        def tail_drain(r, carry):
            pltpu.make_async_copy(
                table_hbm.at[pl.ds(0, 1), pl.ds(_COL7, _TAILW)],
                tails.at[pl.ds(0, 1)],
                sem2,
            ).wait()
            return carry

        lax.fori_loop(0, _CHUNK, tail_drain, 0)

        def tail_compute(r, carry):
            for j in range(_TAILW // _L):
                x = tails[r, pl.ds(j * _L, _L)]
                out_v[r, pl.ds(_COL7 + j * _L, _L)] = _sigmoid(x)
            x = tails[r, pl.ds(_TAILW - _L, _L)]
            out_v[r, pl.ds(NUM_API - _L, _L)] = _sigmoid(x)
            return carry

        lax.fori_loop(0, _CHUNK, tail_compute, 0)
        pltpu.sync_copy(out_v, out_hbm.at[pl.ds(base + c * _CHUNK, _CHUNK)])


def kernel(m2a_mat, user_indices):
    f = pl.kernel(
        _sc_body,
        mesh=plsc.VectorSubcoreMesh(core_axis_name="c", subcore_axis_name="s"),
        out_type=jax.ShapeDtypeStruct((BATCH, NUM_API), jnp.float32),
        scratch_types=[
            pltpu.VMEM((_NCHUNK, _CHUNK), jnp.int32),
            pltpu.VMEM((_CHUNK, _COL7), jnp.float32),
            pltpu.VMEM((_CHUNK, _TAILW), jnp.float32),
            pltpu.VMEM((_CHUNK, NUM_API), jnp.float32),
            pltpu.SemaphoreType.DMA,
            pltpu.SemaphoreType.DMA,
        ],
        compiler_params=pltpu.CompilerParams(needs_layout_passes=False),
    )
    return f(m2a_mat, user_indices.astype(jnp.int32))


# speedup vs baseline: 3.9967x; 1.0109x over previous
import jax
import jax.numpy as jnp
from jax import lax
from jax.experimental import pallas as pl
from jax.experimental.pallas import tpu as pltpu
from jax.experimental.pallas import tpu_sc as plsc

NUM_MASHUP = 100000
NUM_API = 1000
BATCH = 4096

_info = plsc.get_sparse_core_info()
_NC, _NS, _L = _info.num_cores, _info.num_subcores, _info.num_lanes
_NW = _NC * _NS
_BPW = BATCH // _NW
_CHUNK = 32
_NCHUNK = _BPW // _CHUNK
_COL7 = 896
_TAILW = NUM_API - _COL7


def _sigmoid(x):
    return 1.0 / (1.0 + jnp.exp(-x))


def _sc_body(table_hbm, idx_hbm, out_hbm,
             idx_v, in_v0, in_v1, out_v, tails,
             semg0, semg1, semt0, semt1):
    in_v = (in_v0, in_v1)
    semg = (semg0, semg1)
    semt = (semt0, semt1)
    wid = lax.axis_index("s") * _NC + lax.axis_index("c")
    base = wid * _BPW
    lanes = lax.iota(jnp.int32, _L)
    pltpu.sync_copy(idx_hbm.at[pl.ds(base, _BPW)], idx_v)

    def issue_gathers(c, s):
        return [
            pltpu.async_copy(
                table_hbm.at[idx_v.at[pl.ds(c * _CHUNK, _CHUNK)], pl.ds(k * 128, 128)],
                in_v[s].at[:, pl.ds(k * 128, 128)],
                semg[s],
            )
            for k in range(_COL7 // 128)
        ]

    def issue_tails(c, s):
        def body(r, carry):
            grow = c * _CHUNK + r
            voff = pl.multiple_of((grow // _L) * _L, _L)
            v = idx_v[pl.ds(voff, _L)]
            i = jnp.max(jnp.where(lanes == grow % _L, v, 0))
            pltpu.async_copy(
                table_hbm.at[pl.ds(i, 1), pl.ds(_COL7, _TAILW)],
                tails.at[pl.ds(grow, 1)],
                semt[s],
            )
            return carry

        lax.fori_loop(0, _CHUNK, body, 0)

    def wait_tails(s):
        def body(r, carry):
            pltpu.make_async_copy(
                table_hbm.at[pl.ds(0, 1), pl.ds(_COL7, _TAILW)],
                tails.at[pl.ds(0, 1)],
                semt[s],
            ).wait()
            return carry

        lax.fori_loop(0, _CHUNK, body, 0)

    def main_compute(s):
        def body(r, carry):
            for j in range(_COL7 // _L):
                x = in_v[s][r, pl.ds(j * _L, _L)]
                out_v[r, pl.ds(j * _L, _L)] = _sigmoid(x)
            return carry

        lax.fori_loop(0, _CHUNK, body, 0)

    def tail_compute(c, s):
        def body(r, carry):
            grow = c * _CHUNK + r
            for j in range(_TAILW // _L):
                x = tails[grow, pl.ds(j * _L, _L)]
                out_v[r, pl.ds(_COL7 + j * _L, _L)] = _sigmoid(x)
            x = tails[grow, pl.ds(_TAILW - _L, _L)]
            out_v[r, pl.ds(NUM_API - _L, _L)] = _sigmoid(x)
            return carry

        lax.fori_loop(0, _CHUNK, body, 0)

    glist = [issue_gathers(0, 0), issue_gathers(1, 1)]
    issue_tails(0, 0)
    issue_tails(1, 1)
    for c in range(_NCHUNK):
        s = c % 2
        for cp in glist[s]:
            cp.wait()
        main_compute(s)
        wait_tails(s)
        tail_compute(c, s)
        if c + 2 < _NCHUNK:
            glist[s] = issue_gathers(c + 2, s)
            issue_tails(c + 2, s)
        pltpu.sync_copy(out_v, out_hbm.at[pl.ds(base + c * _CHUNK, _CHUNK)])


def kernel(m2a_mat, user_indices):
    f = pl.kernel(
        _sc_body,
        mesh=plsc.VectorSubcoreMesh(core_axis_name="c", subcore_axis_name="s"),
        out_type=jax.ShapeDtypeStruct((BATCH, NUM_API), jnp.float32),
        scratch_types=[
            pltpu.VMEM((_BPW,), jnp.int32),
            pltpu.VMEM((_CHUNK, _COL7), jnp.float32),
            pltpu.VMEM((_CHUNK, _COL7), jnp.float32),
            pltpu.VMEM((_CHUNK, NUM_API), jnp.float32),
            pltpu.VMEM((_BPW, _TAILW), jnp.float32),
            pltpu.SemaphoreType.DMA,
            pltpu.SemaphoreType.DMA,
            pltpu.SemaphoreType.DMA,
            pltpu.SemaphoreType.DMA,
        ],
        compiler_params=pltpu.CompilerParams(needs_layout_passes=False),
    )
    return f(m2a_mat, user_indices.astype(jnp.int32))


# speedup vs baseline: 4.1157x; 1.0298x over previous
import jax
import jax.numpy as jnp
from jax import lax
from jax.experimental import pallas as pl
from jax.experimental.pallas import tpu as pltpu
from jax.experimental.pallas import tpu_sc as plsc

NUM_MASHUP = 100000
NUM_API = 1000
BATCH = 4096

_info = plsc.get_sparse_core_info()
_NC, _NS, _L = _info.num_cores, _info.num_subcores, _info.num_lanes
_NW = _NC * _NS
_BPW = BATCH // _NW
_CHUNK = 32
_NCHUNK = _BPW // _CHUNK
_COL7 = 896
_TAILW = NUM_API - _COL7


def _sigmoid(x):
    return 1.0 / (1.0 + jnp.exp(-x))


def _sc_body(table_hbm, idx_hbm, out_hbm,
             idx_v, in_v0, in_v1, out_v, tails,
             semg0, semg1, semt0, semt1):
    in_v = (in_v0, in_v1)
    semg = (semg0, semg1)
    semt = (semt0, semt1)
    wid = lax.axis_index("s") * _NC + lax.axis_index("c")
    base = wid * _BPW
    lanes = lax.iota(jnp.int32, _L)
    pltpu.sync_copy(idx_hbm.at[pl.ds(base, _BPW)], idx_v)

    def issue_gathers(c, s):
        return [
            pltpu.async_copy(
                table_hbm.at[idx_v.at[pl.ds(c * _CHUNK, _CHUNK)], pl.ds(k * 128, 128)],
                in_v[s].at[:, pl.ds(k * 128, 128)],
                semg[s],
            )
            for k in range(_COL7 // 128)
        ]

    def issue_tails(c, s):
        def body(r, carry):
            grow = c * _CHUNK + r
            voff = pl.multiple_of((grow // _L) * _L, _L)
            v = idx_v[pl.ds(voff, _L)]
            i = jnp.max(jnp.where(lanes == grow % _L, v, 0))
            pltpu.async_copy(
                table_hbm.at[pl.ds(i, 1), pl.ds(_COL7, _TAILW)],
                tails.at[pl.ds(grow, 1)],
                semt[s],
            )
            return carry

        lax.fori_loop(0, _CHUNK, body, 0)

    def wait_tails(s):
        def body(r, carry):
            pltpu.make_async_copy(
                table_hbm.at[pl.ds(0, 1), pl.ds(_COL7, _TAILW)],
                tails.at[pl.ds(0, 1)],
                semt[s],
            ).wait()
            return carry

        lax.fori_loop(0, _CHUNK, body, 0)

    def main_compute(s):
        @plsc.parallel_loop(0, _CHUNK, 1, unroll=2)
        def body(r):
            for j in range(_COL7 // _L):
                x = in_v[s][r, pl.ds(j * _L, _L)]
                out_v[r, pl.ds(j * _L, _L)] = _sigmoid(x)

    def tail_compute(c, s):
        @plsc.parallel_loop(0, _CHUNK, 1, unroll=2)
        def body(r):
            grow = c * _CHUNK + r
            for j in range(_TAILW // _L):
                x = tails[grow, pl.ds(j * _L, _L)]
                out_v[r, pl.ds(_COL7 + j * _L, _L)] = _sigmoid(x)
            x = tails[grow, pl.ds(_TAILW - _L, _L)]
            out_v[r, pl.ds(NUM_API - _L, _L)] = _sigmoid(x)

    glist = [issue_gathers(0, 0), issue_gathers(1, 1)]
    issue_tails(0, 0)
    issue_tails(1, 1)
    for c in range(_NCHUNK):
        s = c % 2
        for cp in glist[s]:
            cp.wait()
        main_compute(s)
        wait_tails(s)
        tail_compute(c, s)
        if c + 2 < _NCHUNK:
            glist[s] = issue_gathers(c + 2, s)
            issue_tails(c + 2, s)
        pltpu.sync_copy(out_v, out_hbm.at[pl.ds(base + c * _CHUNK, _CHUNK)])


def kernel(m2a_mat, user_indices):
    f = pl.kernel(
        _sc_body,
        mesh=plsc.VectorSubcoreMesh(core_axis_name="c", subcore_axis_name="s"),
        out_type=jax.ShapeDtypeStruct((BATCH, NUM_API), jnp.float32),
        scratch_types=[
            pltpu.VMEM((_BPW,), jnp.int32),
            pltpu.VMEM((_CHUNK, _COL7), jnp.float32),
            pltpu.VMEM((_CHUNK, _COL7), jnp.float32),
            pltpu.VMEM((_CHUNK, NUM_API), jnp.float32),
            pltpu.VMEM((_BPW, _TAILW), jnp.float32),
            pltpu.SemaphoreType.DMA,
            pltpu.SemaphoreType.DMA,
            pltpu.SemaphoreType.DMA,
            pltpu.SemaphoreType.DMA,
        ],
        compiler_params=pltpu.CompilerParams(needs_layout_passes=False),
    )
    return f(m2a_mat, user_indices.astype(jnp.int32))


# speedup vs baseline: 5.3550x; 1.3011x over previous
import jax
import jax.numpy as jnp
from jax import lax
from jax.experimental import pallas as pl
from jax.experimental.pallas import tpu as pltpu
from jax.experimental.pallas import tpu_sc as plsc

NUM_MASHUP = 100000
NUM_API = 1000
BATCH = 4096

_info = plsc.get_sparse_core_info()
_NC, _NS, _L = _info.num_cores, _info.num_subcores, _info.num_lanes
_NW = _NC * _NS
_BPW = BATCH // _NW
_CHUNK = 32
_NCHUNK = _BPW // _CHUNK
_COL7 = 896
_TAILW = NUM_API - _COL7


def _sigmoid(x):
    return 1.0 / (1.0 + jnp.exp(-x))


def _sc_body(table_hbm, idx_hbm, out_hbm,
             idx_v, in_v0, in_v1, out_v, tails,
             semg0, semg1, semt0, semt1):
    in_v = (in_v0, in_v1)
    semg = (semg0, semg1)
    semt = (semt0, semt1)
    wid = lax.axis_index("s") * _NC + lax.axis_index("c")
    base = wid * _BPW
    lanes = lax.iota(jnp.int32, _L)
    pltpu.sync_copy(idx_hbm.at[pl.ds(base, _BPW)], idx_v)

    def issue_gathers(c, s):
        return [
            pltpu.async_copy(
                table_hbm.at[idx_v.at[pl.ds(c * _CHUNK, _CHUNK)], pl.ds(k * 128, 128)],
                in_v[s].at[:, pl.ds(k * 128, 128)],
                semg[s],
            )
            for k in range(_COL7 // 128)
        ]

    def issue_tails(c, s):
        def body(r, carry):
            grow = c * _CHUNK + r
            voff = pl.multiple_of((grow // _L) * _L, _L)
            v = idx_v[pl.ds(voff, _L)]
            i = jnp.max(jnp.where(lanes == grow % _L, v, 0))
            pltpu.async_copy(
                table_hbm.at[pl.ds(i, 1), pl.ds(_COL7, _TAILW)],
                tails.at[pl.ds(grow, 1)],
                semt[s],
            )
            return carry

        lax.fori_loop(0, _CHUNK, body, 0)

    def wait_tails(s):
        def body(r, carry):
            pltpu.make_async_copy(
                table_hbm.at[pl.ds(0, 1), pl.ds(_COL7, _TAILW)],
                tails.at[pl.ds(0, 1)],
                semt[s],
            ).wait()
            return carry

        lax.fori_loop(0, _CHUNK, body, 0)

    def main_compute(s):
        def body(r, carry):
            @plsc.parallel_loop(0, _COL7 // _L, 1, unroll=8)
            def jloop(j):
                o = pl.multiple_of(j * _L, _L)
                x = in_v[s][r, pl.ds(o, _L)]
                out_v[r, pl.ds(o, _L)] = _sigmoid(x)
            return carry

        lax.fori_loop(0, _CHUNK, body, 0)

    def tail_compute(c, s):
        @plsc.parallel_loop(0, _CHUNK, 1, unroll=2)
        def body(r):
            grow = c * _CHUNK + r
            for j in range(_TAILW // _L):
                x = tails[grow, pl.ds(j * _L, _L)]
                out_v[r, pl.ds(_COL7 + j * _L, _L)] = _sigmoid(x)
            x = tails[grow, pl.ds(_TAILW - _L, _L)]
            out_v[r, pl.ds(NUM_API - _L, _L)] = _sigmoid(x)

    glist = [issue_gathers(0, 0), issue_gathers(1, 1)]
    issue_tails(0, 0)
    issue_tails(1, 1)
    for c in range(_NCHUNK):
        s = c % 2
        for cp in glist[s]:
            cp.wait()
        main_compute(s)
        wait_tails(s)
        tail_compute(c, s)
        if c + 2 < _NCHUNK:
            glist[s] = issue_gathers(c + 2, s)
            issue_tails(c + 2, s)
        pltpu.sync_copy(out_v, out_hbm.at[pl.ds(base + c * _CHUNK, _CHUNK)])


def kernel(m2a_mat, user_indices):
    f = pl.kernel(
        _sc_body,
        mesh=plsc.VectorSubcoreMesh(core_axis_name="c", subcore_axis_name="s"),
        out_type=jax.ShapeDtypeStruct((BATCH, NUM_API), jnp.float32),
        scratch_types=[
            pltpu.VMEM((_BPW,), jnp.int32),
            pltpu.VMEM((_CHUNK, _COL7), jnp.float32),
            pltpu.VMEM((_CHUNK, _COL7), jnp.float32),
            pltpu.VMEM((_CHUNK, NUM_API), jnp.float32),
            pltpu.VMEM((_BPW, _TAILW), jnp.float32),
            pltpu.SemaphoreType.DMA,
            pltpu.SemaphoreType.DMA,
            pltpu.SemaphoreType.DMA,
            pltpu.SemaphoreType.DMA,
        ],
        compiler_params=pltpu.CompilerParams(needs_layout_passes=False),
    )
    return f(m2a_mat, user_indices.astype(jnp.int32))
